# baseline (speedup 1.0000x reference)
import functools

import jax
import jax.numpy as jnp
from jax import lax
from jax.experimental import pallas as pl
from jax.experimental.pallas import tpu as pltpu
from jax.experimental.pallas import tpu_sc as plsc

N = 2048
D = 128
E = 65536
NW = 32
ROWS = 32
NBLK = N // ROWS
BPW = NBLK // NW
CHUNK = 8192
NCH = E // CHUNK


@functools.cache
def _sc_build_fn():
    mesh = plsc.VectorSubcoreMesh(core_axis_name="c", subcore_axis_name="s")

    @functools.partial(
        pl.kernel,
        out_type=jax.ShapeDtypeStruct((N, N), jnp.float32),
        mesh=mesh,
        compiler_params=pltpu.CompilerParams(needs_layout_passes=False),
        scratch_types=(
            pltpu.VMEM((ROWS, N), jnp.float32),
            pltpu.VMEM((CHUNK,), jnp.int32),
            pltpu.VMEM((CHUNK,), jnp.int32),
            pltpu.VMEM((CHUNK,), jnp.int32),
            pltpu.VMEM((CHUNK,), jnp.int32),
            pltpu.SemaphoreType.DMA,
            pltpu.SemaphoreType.DMA,
        ),
    )
    def _sc_build(dst_hbm, src_hbm, c_hbm, acc,
                  dbuf0, sbuf0, dbuf1, sbuf1, sem0, sem1):
        wid = lax.axis_index("s") * 2 + lax.axis_index("c")
        slots = ((dbuf0, sbuf0, sem0), (dbuf1, sbuf1, sem1))
        ones = jnp.ones((16,), jnp.float32)
        zeros = jnp.zeros((16,), jnp.float32)

        def start(gch, slot):
            ch = gch % NCH
            db, sb, sem = slots[slot]
            return (
                pltpu.async_copy(dst_hbm.at[pl.ds(ch * CHUNK, CHUNK)],
                                 db, sem),
                pltpu.async_copy(src_hbm.at[pl.ds(ch * CHUNK, CHUNK)],
                                 sb, sem),
            )

        pending = {0: start(0, 0)}
        for p in range(BPW):
            blk = wid + NW * p
            base = blk * ROWS

            @plsc.parallel_loop(0, N // 16, unroll=2)
            def zbody(j):
                for r in range(ROWS):
                    acc[r, pl.ds(j * 16, 16)] = zeros

            for ch in range(NCH):
                gch = p * NCH + ch
                slot = gch % 2
                for h in pending.pop(gch):
                    h.wait()
                if gch + 1 < BPW * NCH:
                    pending[gch + 1] = start(gch + 1, 1 - slot)
                db, sb, _ = slots[slot]

                @plsc.parallel_loop(0, CHUNK // 16, unroll=8)
                def ebody(i):
                    dv = db[pl.ds(i * 16, 16)]
                    sv = sb[pl.ds(i * 16, 16)]
                    loc = dv - base
                    m = (loc >= 0) & (loc < ROWS)
                    locc = lax.bitwise_and(loc, ROWS - 1)
                    plsc.addupdate_scatter(acc, [locc, sv], ones, mask=m)

            for h in range(ROWS // 16):
                r = lax.iota(jnp.int32, 16) + h * 16
                plsc.addupdate_scatter(acc, [r, base + r], ones)

            pltpu.sync_copy(acc, c_hbm.at[pl.ds(base, ROWS)])

    return _sc_build


BR = 256
_PREC = lax.Precision.HIGHEST


def _tc_body(x_ref, w0_ref, c_ref, w1_ref, b0_ref, b1_ref, out_ref,
             c_scr, deg_scr, t0_scr, t1_scr):
    p = pl.program_id(0)
    g = pl.program_id(1)
    rows = pl.ds(g * BR, BR)

    @pl.when(p == 0)
    def _():
        cb = c_ref[...]
        c_scr[rows, :] = cb
        deg_scr[rows, :] = jnp.sum(cb, axis=1, keepdims=True)

    @pl.when((p == 1) & (g == 0))
    def _():
        t0_scr[...] = lax.rsqrt(deg_scr[...]) * jnp.dot(
            x_ref[...], w0_ref[...],
            preferred_element_type=jnp.float32, precision=_PREC)

    @pl.when(p == 1)
    def _():
        disb = lax.rsqrt(deg_scr[rows, :])
        m = jnp.dot(c_scr[rows, :], t0_scr[...],
                    preferred_element_type=jnp.float32)
        h1 = jnp.maximum(disb * m + b0_ref[...], 0.0)
        t1_scr[rows, :] = jnp.dot(
            h1, w1_ref[...],
            preferred_element_type=jnp.float32, precision=_PREC) * disb

    @pl.when(p == 2)
    def _():
        disb = lax.rsqrt(deg_scr[rows, :])
        out_ref[...] = disb * jnp.dot(
            c_scr[rows, :], t1_scr[...],
            preferred_element_type=jnp.float32) + b1_ref[...]


def _gcn_stack(x, C, W0, b0, W1, b1):
    full = lambda p, g: (0, 0)
    return pl.pallas_call(
        _tc_body,
        grid=(3, N // BR),
        in_specs=[
            pl.BlockSpec((N, D), full),
            pl.BlockSpec((D, D), full),
            pl.BlockSpec((BR, N), lambda p, g: (jnp.where(p == 0, g, 0), 0)),
            pl.BlockSpec((D, D), full),
            pl.BlockSpec((1, D), full),
            pl.BlockSpec((1, D), full),
        ],
        out_specs=pl.BlockSpec((BR, D), lambda p, g: (g, 0)),
        out_shape=jax.ShapeDtypeStruct((N, D), jnp.float32),
        scratch_shapes=[
            pltpu.VMEM((N, N), jnp.float32),
            pltpu.VMEM((N, 1), jnp.float32),
            pltpu.VMEM((N, D), jnp.float32),
            pltpu.VMEM((N, D), jnp.float32),
        ],
        compiler_params=pltpu.CompilerParams(
            vmem_limit_bytes=100 * 1024 * 1024),
    )(x, W0, C, W1, b0.reshape(1, D), b1.reshape(1, D))


def kernel(x, edge_index, coordinates, W0, b0, W1, b1):
    ei = edge_index.astype(jnp.int32)
    src = ei[0]
    dst = ei[1]
    C = _sc_build_fn()(dst, src)
    out = _gcn_stack(x, C, W0, b0, W1, b1)
    labels = jax.random.randint(jax.random.key(42), (x.shape[0],), 0, 3)
    return (out, labels)


# speedup vs baseline: 1.0752x; 1.0752x over previous
import functools

import jax
import jax.numpy as jnp
from jax import lax
from jax.experimental import pallas as pl
from jax.experimental.pallas import tpu as pltpu
from jax.experimental.pallas import tpu_sc as plsc

N = 2048
D = 128
E = 65536
NW = 32
ROWS = 32
NBLK = N // ROWS
BPW = NBLK // NW
CHUNK = 8192
NCH = E // CHUNK


@functools.cache
def _sc_build_fn():
    mesh = plsc.VectorSubcoreMesh(core_axis_name="c", subcore_axis_name="s")

    @functools.partial(
        pl.kernel,
        out_type=jax.ShapeDtypeStruct((N, N), jnp.float32),
        mesh=mesh,
        compiler_params=pltpu.CompilerParams(needs_layout_passes=False),
        scratch_types=(
            pltpu.VMEM((ROWS, N), jnp.float32),
            pltpu.VMEM((CHUNK,), jnp.int32),
            pltpu.VMEM((CHUNK,), jnp.int32),
            pltpu.VMEM((CHUNK,), jnp.int32),
            pltpu.VMEM((CHUNK,), jnp.int32),
            pltpu.SemaphoreType.DMA,
            pltpu.SemaphoreType.DMA,
        ),
    )
    def _sc_build(dst_hbm, src_hbm, c_hbm, acc,
                  dbuf0, sbuf0, dbuf1, sbuf1, sem0, sem1):
        wid = lax.axis_index("s") * 2 + lax.axis_index("c")
        slots = ((dbuf0, sbuf0, sem0), (dbuf1, sbuf1, sem1))
        ones = jnp.ones((16,), jnp.float32)
        zeros = jnp.zeros((16,), jnp.float32)

        def start(gch, slot):
            ch = gch % NCH
            db, sb, sem = slots[slot]
            return (
                pltpu.async_copy(dst_hbm.at[pl.ds(ch * CHUNK, CHUNK)],
                                 db, sem),
                pltpu.async_copy(src_hbm.at[pl.ds(ch * CHUNK, CHUNK)],
                                 sb, sem),
            )

        pending = {0: start(0, 0)}
        for p in range(BPW):
            blk = wid + NW * p
            base = blk * ROWS

            @plsc.parallel_loop(0, N // 16, unroll=2)
            def zbody(j):
                for r in range(ROWS):
                    acc[r, pl.ds(j * 16, 16)] = zeros

            for ch in range(NCH):
                gch = p * NCH + ch
                slot = gch % 2
                for h in pending.pop(gch):
                    h.wait()
                if gch + 1 < BPW * NCH:
                    pending[gch + 1] = start(gch + 1, 1 - slot)
                db, sb, _ = slots[slot]

                @plsc.parallel_loop(0, CHUNK // 16, unroll=8)
                def ebody(i):
                    dv = db[pl.ds(i * 16, 16)]
                    sv = sb[pl.ds(i * 16, 16)]
                    loc = dv - base
                    m = (loc >= 0) & (loc < ROWS)
                    locc = lax.bitwise_and(loc, ROWS - 1)
                    plsc.addupdate_scatter(acc, [locc, sv], ones, mask=m)

            for h in range(ROWS // 16):
                r = lax.iota(jnp.int32, 16) + h * 16
                plsc.addupdate_scatter(acc, [r, base + r], ones)

            pltpu.sync_copy(acc, c_hbm.at[pl.ds(base, ROWS)])

    return _sc_build


BR = 256
_PREC = lax.Precision.HIGHEST


def _tc_body(x_ref, w0_ref, c_ref, w1_ref, b0_ref, b1_ref, out_ref):
    deg = jnp.sum(c_ref[...], axis=1, keepdims=True)
    dis = lax.rsqrt(deg)
    t0 = dis * jnp.dot(x_ref[...], w0_ref[...],
                       preferred_element_type=jnp.float32, precision=_PREC)
    m = jnp.dot(c_ref[...], t0, preferred_element_type=jnp.float32)
    h1 = jnp.maximum(dis * m + b0_ref[...], 0.0)
    t1 = jnp.dot(h1, w1_ref[...],
                 preferred_element_type=jnp.float32, precision=_PREC) * dis
    out_ref[...] = dis * jnp.dot(
        c_ref[...], t1, preferred_element_type=jnp.float32) + b1_ref[...]


def _gcn_stack(x, C, W0, b0, W1, b1):
    return pl.pallas_call(
        _tc_body,
        out_shape=jax.ShapeDtypeStruct((N, D), jnp.float32),
        compiler_params=pltpu.CompilerParams(
            vmem_limit_bytes=100 * 1024 * 1024),
    )(x, W0, C, W1, b0.reshape(1, D), b1.reshape(1, D))


def kernel(x, edge_index, coordinates, W0, b0, W1, b1):
    ei = edge_index.astype(jnp.int32)
    src = ei[0]
    dst = ei[1]
    C = _sc_build_fn()(dst, src)
    out = _gcn_stack(x, C, W0, b0, W1, b1)
    labels = jax.random.randint(jax.random.key(42), (x.shape[0],), 0, 3)
    return (out, labels)
